# Initial kernel scaffold; baseline (speedup 1.0000x reference)
import jax
import jax.numpy as jnp
from jax.experimental import pallas as pl


def kernel(x, edge_index, edge_attr, W_t, W_edge, nu_kt, nu_edge):
    raise NotImplementedError("write your pallas kernel here")


# speedup vs baseline: 8.2471x; 8.2471x over previous
import functools

import jax
import jax.numpy as jnp
from jax import lax
from jax.experimental import pallas as pl
from jax.experimental.pallas import tpu as pltpu
from jax.experimental.pallas import tpu_sc as plsc

N = 10000
E = 320000
D = 128

NC = 2
NS = 16
NW = NC * NS
L = 16

EPT = E // NW
CH = 80
NCHUNK = EPT // CH

ACC_ROWS = 10240
RPT = ACC_ROWS // NS
ZB = 8
WB = 80


def _sc_pass(with_hop, table, src, dst, et, hop):
    mesh = plsc.VectorSubcoreMesh(core_axis_name="c", subcore_axis_name="s")

    def body(table_h, src_h, dst_h, et_h, hop_h, zeros_h, out_h,
             src_v, et_v, hop_v, di_v, di2_v, gi_v, gi2_v,
             rows_v, rows2_v, st_v, acc_sh, sem, sem2):
        c = lax.axis_index("c")
        s = lax.axis_index("s")
        wid = c * NS + s

        pltpu.sync_copy(zeros_h, st_v)

        def zero_body(j, _):
            pltpu.sync_copy(st_v, acc_sh.at[pl.ds(s * RPT + j * WB, WB)])
            return 0
        lax.fori_loop(0, RPT // WB, zero_body, 0)
        plsc.subcore_barrier()

        trash = ACC_ROWS - NS * ZB + s * ZB

        def chunk_body(j, _):
            base = pl.multiple_of(wid * EPT + j * CH, 8)
            pltpu.sync_copy(src_h.at[pl.ds(base, CH)], src_v)
            pltpu.sync_copy(et_h.at[pl.ds(base, CH)], et_v)
            pltpu.sync_copy(dst_h.at[pl.ds(base, CH)], di_v)
            if with_hop:
                pltpu.sync_copy(hop_h.at[pl.ds(base, CH)], hop_v)
            for q in range(CH // L):
                sl = pl.ds(q * L, L)
                sv = src_v[sl]
                gi_v[sl] = et_v[sl] * N + sv
                if with_hop:
                    gi2_v[sl] = sv + 3 * N
                    di2_v[sl] = jnp.where(hop_v[sl] == 2, di_v[sl], trash)
            cp = pltpu.async_copy(table_h.at[gi_v], rows_v, sem)
            if with_hop:
                cp2 = pltpu.async_copy(table_h.at[gi2_v], rows2_v, sem2)
            cp.wait()
            pltpu.sync_copy(rows_v, acc_sh.at[di_v], add=True)
            if with_hop:
                cp2.wait()
                pltpu.sync_copy(rows2_v, acc_sh.at[di2_v], add=True)
            return 0
        lax.fori_loop(0, NCHUNK, chunk_body, 0)
        plsc.subcore_barrier()

        def out_body(j, _):
            r = s * RPT + j * WB
            pltpu.sync_copy(acc_sh.at[pl.ds(r, WB)], st_v)
            pltpu.sync_copy(st_v, out_h.at[pl.ds(c * ACC_ROWS + r, WB)])
            return 0
        lax.fori_loop(0, RPT // WB, out_body, 0)

    k = pl.kernel(
        body,
        out_type=jax.ShapeDtypeStruct((NC * ACC_ROWS, D), jnp.float32),
        mesh=mesh,
        scratch_types=[
            pltpu.VMEM((CH,), jnp.int32),
            pltpu.VMEM((CH,), jnp.int32),
            pltpu.VMEM((CH,), jnp.int32),
            pltpu.VMEM((CH,), jnp.int32),
            pltpu.VMEM((CH,), jnp.int32),
            pltpu.VMEM((CH,), jnp.int32),
            pltpu.VMEM((CH,), jnp.int32),
            pltpu.VMEM((CH, D), jnp.float32),
            pltpu.VMEM((CH, D), jnp.float32),
            pltpu.VMEM((WB, D), jnp.float32),
            pltpu.VMEM_SHARED((ACC_ROWS, D), jnp.float32),
            pltpu.SemaphoreType.DMA,
            pltpu.SemaphoreType.DMA,
        ],
    )
    zeros = jnp.zeros((WB, D), jnp.float32)
    return k(table, src, dst, et, hop, zeros).reshape(NC, ACC_ROWS, D)


NB = 10
BR = N // NB


def _table0_kernel(x_ref, w_ref, t_ref):
    for e in range(4):
        t_ref[e] = jnp.dot(x_ref[...], w_ref[e],
                           preferred_element_type=jnp.float32)


def _mk_table0(x, w4):
    return pl.pallas_call(
        _table0_kernel,
        grid=(NB,),
        in_specs=[
            pl.BlockSpec((BR, D), lambda i: (i, 0)),
            pl.BlockSpec((4, D, D), lambda i: (0, 0, 0)),
        ],
        out_specs=pl.BlockSpec((4, BR, D), lambda i: (0, i, 0)),
        out_shape=jax.ShapeDtypeStruct((4, N, D), jnp.float32),
    )(x, w4)


def _update_kernel(make_table, x_ref, p_ref, w_ref, t03_ref, cur_ref, t_ref):
    a = p_ref[0] + p_ref[1]
    h = x_ref[...] + jnp.maximum(a, 0.0)
    nrm = jnp.sqrt(jnp.sum(h * h, axis=1, keepdims=True))
    cur = h / jnp.maximum(nrm, 1e-12)
    cur_ref[...] = cur
    if make_table:
        for e in range(3):
            t_ref[e] = jnp.dot(cur, w_ref[e],
                               preferred_element_type=jnp.float32)
        t_ref[3] = t03_ref[0]


def _mk_update(x, parts, w3, t03):
    return pl.pallas_call(
        functools.partial(_update_kernel, True),
        grid=(NB,),
        in_specs=[
            pl.BlockSpec((BR, D), lambda i: (i, 0)),
            pl.BlockSpec((NC, BR, D), lambda i: (0, i, 0)),
            pl.BlockSpec((3, D, D), lambda i: (0, 0, 0)),
            pl.BlockSpec((1, BR, D), lambda i: (3, i, 0)),
        ],
        out_specs=[
            pl.BlockSpec((BR, D), lambda i: (i, 0)),
            pl.BlockSpec((4, BR, D), lambda i: (0, i, 0)),
        ],
        out_shape=[
            jax.ShapeDtypeStruct((N, D), jnp.float32),
            jax.ShapeDtypeStruct((4, N, D), jnp.float32),
        ],
    )(x, parts, w3, t03)


def _final_kernel(x_ref, p_ref, cur_ref):
    a = p_ref[0] + p_ref[1]
    h = x_ref[...] + jnp.maximum(a, 0.0)
    nrm = jnp.sqrt(jnp.sum(h * h, axis=1, keepdims=True))
    cur_ref[...] = h / jnp.maximum(nrm, 1e-12)


def _mk_final(x, parts):
    return pl.pallas_call(
        _final_kernel,
        grid=(NB,),
        in_specs=[
            pl.BlockSpec((BR, D), lambda i: (i, 0)),
            pl.BlockSpec((NC, BR, D), lambda i: (0, i, 0)),
        ],
        out_specs=pl.BlockSpec((BR, D), lambda i: (i, 0)),
        out_shape=jax.ShapeDtypeStruct((N, D), jnp.float32),
    )(x, parts)


def kernel(x, edge_index, edge_attr, W_t, W_edge, nu_kt, nu_edge):
    src = edge_index[0]
    dst = edge_index[1]
    hop = edge_attr[:, 0]
    etype = edge_attr[:, 1]

    w0 = jnp.concatenate(
        [W_edge[0] * nu_edge[0, :, None, None],
         (W_t[1] * nu_kt[1, 0])[None]], axis=0)
    w1 = W_edge[1] * nu_edge[1, :, None, None]

    t0 = _mk_table0(x, w0)
    parts0 = _sc_pass(False, t0.reshape(4 * N, D), src, dst, etype, hop)
    cur1, t1 = _mk_update(x, parts0, w1, t0)
    parts1 = _sc_pass(True, t1.reshape(4 * N, D), src, dst, etype, hop)
    return _mk_final(cur1, parts1)


# speedup vs baseline: 8.7121x; 1.0564x over previous
import functools

import jax
import jax.numpy as jnp
from jax import lax
from jax.experimental import pallas as pl
from jax.experimental.pallas import tpu as pltpu
from jax.experimental.pallas import tpu_sc as plsc

N = 10000
E = 320000
D = 128

NC = 2
NS = 16
NW = NC * NS
L = 16

CH = 64
NCHUNK = 158
EPT = CH * NCHUNK
E_PAD = NW * EPT
PK = 4 * CH

ACC_ROWS = 10240
RPT = ACC_ROWS // NS
ZB = 8
PAD_ROW = 10016
WB = 40


def _sc_pass(with_hop, table, packed):
    mesh = plsc.VectorSubcoreMesh(core_axis_name="c", subcore_axis_name="s")
    GL = 2 * CH if with_hop else CH

    def body(table_h, packed_h, zeros_h, out_h,
             pkA, pkB, gA, gB, dA, dB, rA, rB, stA,
             acc_sh,
             semI_A, semI_B, semG_A, semG_B):
        c = lax.axis_index("c")
        s = lax.axis_index("s")
        wid = c * NS + s
        cbase = wid * NCHUNK
        trash = ACC_ROWS - NS * ZB + s * ZB

        SETA = (pkA, gA, dA, rA, semI_A, semG_A)
        SETB = (pkB, gB, dB, rB, semI_B, semG_B)

        def idx_load(j, st):
            pk, _, _, _, semI, _ = st
            off = pl.multiple_of((cbase + j) * PK, 8)
            pltpu.async_copy(packed_h.at[pl.ds(off, PK)], pk, semI)

        def idx_wait(st):
            pk, _, _, _, semI, _ = st
            pltpu.make_async_copy(packed_h.at[pl.ds(0, PK)], pk, semI).wait()

        def idx_compute(st):
            pk, g, d, _, _, _ = st
            for q in range(CH // L):
                sl = pl.ds(q * L, L)
                sv = pk[pl.ds(q * L, L)]
                ev = pk[pl.ds(CH + q * L, L)]
                dv = pk[pl.ds(2 * CH + q * L, L)]
                g[sl] = ev * N + sv
                d[sl] = dv
                if with_hop:
                    sl2 = pl.ds(CH + q * L, L)
                    hv = pk[pl.ds(3 * CH + q * L, L)]
                    g[sl2] = sv + 3 * N
                    d[sl2] = jnp.where(hv == 2, dv, trash)

        def gather_issue(st):
            _, g, _, r, _, semG = st
            pltpu.async_copy(table_h.at[g], r, semG)

        def gather_wait(st):
            _, g, _, r, _, semG = st
            pltpu.make_async_copy(table_h.at[g], r, semG).wait()

        def scatter(st):
            _, _, d, r, _, _ = st
            pltpu.sync_copy(r, acc_sh.at[d], add=True)

        pltpu.sync_copy(zeros_h, stA)

        @pl.loop(0, RPT // WB)
        def zero_body(j):
            pltpu.sync_copy(stA, acc_sh.at[pl.ds(s * RPT + j * WB, WB)])
        plsc.subcore_barrier()

        idx_load(0, SETA)
        idx_wait(SETA)
        idx_compute(SETA)
        gather_issue(SETA)
        idx_load(1, SETB)

        def half(j, cur, nxt):
            idx_wait(nxt)
            idx_compute(nxt)
            gather_wait(cur)
            gather_issue(nxt)
            idx_load(j + 2, cur)
            scatter(cur)

        @pl.loop(0, NCHUNK // 2 - 1)
        def pair_body(m):
            half(2 * m, SETA, SETB)
            half(2 * m + 1, SETB, SETA)
        idx_wait(SETB)
        idx_compute(SETB)
        gather_wait(SETA)
        gather_issue(SETB)
        scatter(SETA)
        gather_wait(SETB)
        scatter(SETB)
        plsc.subcore_barrier()

        @pl.loop(0, RPT // WB)
        def out_body(m):
            r0 = s * RPT + m * WB
            pltpu.sync_copy(acc_sh.at[pl.ds(r0, WB)], stA)
            pltpu.sync_copy(stA, out_h.at[pl.ds(c * ACC_ROWS + r0, WB)])

    k = pl.kernel(
        body,
        out_type=jax.ShapeDtypeStruct((NC * ACC_ROWS, D), jnp.float32),
        mesh=mesh,
        scratch_types=[
            pltpu.VMEM((PK,), jnp.int32),
            pltpu.VMEM((PK,), jnp.int32),
            pltpu.VMEM((GL,), jnp.int32),
            pltpu.VMEM((GL,), jnp.int32),
            pltpu.VMEM((GL,), jnp.int32),
            pltpu.VMEM((GL,), jnp.int32),
            pltpu.VMEM((GL, D), jnp.float32),
            pltpu.VMEM((GL, D), jnp.float32),
            pltpu.VMEM((WB, D), jnp.float32),
            pltpu.VMEM_SHARED((ACC_ROWS, D), jnp.float32),
            pltpu.SemaphoreType.DMA,
            pltpu.SemaphoreType.DMA,
            pltpu.SemaphoreType.DMA,
            pltpu.SemaphoreType.DMA,
        ],
    )
    zeros = jnp.zeros((WB, D), jnp.float32)
    return k(table, packed, zeros).reshape(NC, ACC_ROWS, D)


NB = 10
BR = N // NB


def _table0_kernel(x_ref, w_ref, t_ref):
    for e in range(4):
        t_ref[e] = jnp.dot(x_ref[...], w_ref[e],
                           preferred_element_type=jnp.float32)


def _mk_table0(x, w4):
    return pl.pallas_call(
        _table0_kernel,
        grid=(NB,),
        in_specs=[
            pl.BlockSpec((BR, D), lambda i: (i, 0)),
            pl.BlockSpec((4, D, D), lambda i: (0, 0, 0)),
        ],
        out_specs=pl.BlockSpec((4, BR, D), lambda i: (0, i, 0)),
        out_shape=jax.ShapeDtypeStruct((4, N, D), jnp.float32),
    )(x, w4)


def _update_kernel(make_table, x_ref, p_ref, w_ref, t03_ref, cur_ref, t_ref):
    a = p_ref[0] + p_ref[1]
    h = x_ref[...] + jnp.maximum(a, 0.0)
    nrm = jnp.sqrt(jnp.sum(h * h, axis=1, keepdims=True))
    cur = h / jnp.maximum(nrm, 1e-12)
    cur_ref[...] = cur
    if make_table:
        for e in range(3):
            t_ref[e] = jnp.dot(cur, w_ref[e],
                               preferred_element_type=jnp.float32)
        t_ref[3] = t03_ref[0]


def _mk_update(x, parts, w3, t03):
    return pl.pallas_call(
        functools.partial(_update_kernel, True),
        grid=(NB,),
        in_specs=[
            pl.BlockSpec((BR, D), lambda i: (i, 0)),
            pl.BlockSpec((NC, BR, D), lambda i: (0, i, 0)),
            pl.BlockSpec((3, D, D), lambda i: (0, 0, 0)),
            pl.BlockSpec((1, BR, D), lambda i: (3, i, 0)),
        ],
        out_specs=[
            pl.BlockSpec((BR, D), lambda i: (i, 0)),
            pl.BlockSpec((4, BR, D), lambda i: (0, i, 0)),
        ],
        out_shape=[
            jax.ShapeDtypeStruct((N, D), jnp.float32),
            jax.ShapeDtypeStruct((4, N, D), jnp.float32),
        ],
    )(x, parts, w3, t03)


def _final_kernel(x_ref, p_ref, cur_ref):
    a = p_ref[0] + p_ref[1]
    h = x_ref[...] + jnp.maximum(a, 0.0)
    nrm = jnp.sqrt(jnp.sum(h * h, axis=1, keepdims=True))
    cur_ref[...] = h / jnp.maximum(nrm, 1e-12)


def _mk_final(x, parts):
    return pl.pallas_call(
        _final_kernel,
        grid=(NB,),
        in_specs=[
            pl.BlockSpec((BR, D), lambda i: (i, 0)),
            pl.BlockSpec((NC, BR, D), lambda i: (0, i, 0)),
        ],
        out_specs=pl.BlockSpec((BR, D), lambda i: (i, 0)),
        out_shape=jax.ShapeDtypeStruct((N, D), jnp.float32),
    )(x, parts)


def kernel(x, edge_index, edge_attr, W_t, W_edge, nu_kt, nu_edge):
    src = edge_index[0]
    dst = edge_index[1]
    hop = edge_attr[:, 0]
    etype = edge_attr[:, 1]

    pad = E_PAD - E
    zpad = jnp.zeros((pad,), jnp.int32)
    src_p = jnp.concatenate([src, zpad])
    et_p = jnp.concatenate([etype, zpad])
    hop_p = jnp.concatenate([hop, zpad])
    dst_p = jnp.concatenate(
        [dst, PAD_ROW + (jnp.arange(pad, dtype=jnp.int32) % 64)])

    packed = jnp.concatenate(
        [src_p.reshape(-1, CH), et_p.reshape(-1, CH),
         dst_p.reshape(-1, CH), hop_p.reshape(-1, CH)], axis=1).reshape(-1)

    w0 = jnp.concatenate(
        [W_edge[0] * nu_edge[0, :, None, None],
         (W_t[1] * nu_kt[1, 0])[None]], axis=0)
    w1 = W_edge[1] * nu_edge[1, :, None, None]

    t0 = _mk_table0(x, w0)
    parts0 = _sc_pass(False, t0.reshape(4 * N, D), packed)
    cur1, t1 = _mk_update(x, parts0, w1, t0)
    parts1 = _sc_pass(True, t1.reshape(4 * N, D), packed)
    return _mk_final(cur1, parts1)
